# baseline (speedup 1.0000x reference)
import functools

import jax
import jax.numpy as jnp
import numpy as np
from jax import lax
from jax.experimental import pallas as pl
from jax.experimental.pallas import tpu as pltpu
from jax.experimental.pallas import tpu_sc as plsc

_P = 8
_N = 100
_T = 10
_W20 = 2 * _T
_TWO_PI = np.float32(2.0 * np.pi)
_PI = np.float32(np.pi)
_HALF_PI = np.float32(np.pi / 2.0)
_Q = np.float32((2.0 * np.pi) / _P)

_ATAN_COEF = [np.float32(c) for c in (
    0.9999998999652773, -0.33332674305713506, 0.19987152762793234,
    -0.1417006414600397, 0.10531652562455919, -0.07302710404060626,
    0.040575162432038944, -0.01489037185530793, 0.0025799282931876833,
)]

_NC = 2
_NS = 16
_NW = _NC * _NS
_L = 16

_A = 64


def _atan2(a, b):
    aa = jnp.abs(a)
    ab = jnp.abs(b)
    mx = jnp.maximum(aa, ab)
    mn = jnp.minimum(aa, ab)
    den = jnp.where(mx > 0, mx, jnp.float32(1.0))
    t = mn / den
    z = t * t
    acc = _ATAN_COEF[-1]
    for c in _ATAN_COEF[-2::-1]:
        acc = acc * z + c
    r = t * acc
    r = jnp.where(aa > ab, _HALF_PI - r, r)
    r = jnp.where(b < 0, _PI - r, r)
    r = jnp.where(a < 0, -r, r)
    return r


def _feat_body(x_ref, vel_ref, dist_ref, th_ref, bin_ref):
    x = x_ref[...]
    r = lax.broadcasted_iota(jnp.int32, (_N * _W20, 640), 0)
    c = lax.broadcasted_iota(jnp.int32, (_N * _W20, 640), 1)
    g = c // 128
    j = c % 128
    i0 = j * _W20
    ssum = ((g == 0) & (r // _W20 == j)).astype(jnp.float32)
    spos = (((g == 1) & (r == i0 + 18)) |
            ((g == 2) & (r == i0 + 19)) |
            ((g == 3) & (r == i0 + 18)) |
            ((g == 4) & (r == i0 + 19))).astype(jnp.float32)
    sneg = (((g == 1) & (r == i0)) |
            ((g == 2) & (r == i0 + 1))).astype(jnp.float32)
    sel = jnp.where(j < _N, ssum + spos - sneg, jnp.float32(0.0))
    y = lax.dot_general(x, sel, (((1,), (0,)), ((), ())),
                        preferred_element_type=jnp.float32)
    sums = y[:, 0:_N]
    vx = y[:, 128:128 + _N]
    vy = y[:, 256:256 + _N]
    lx = y[:, 384:384 + _N]
    ly = y[:, 512:512 + _N]
    vel = jnp.sqrt(vx * vx + vy * vy)
    dist = jnp.sqrt(lx * lx + ly * ly)
    th = _atan2(lx, ly)
    th2 = jnp.where(th < 0, th + _TWO_PI, th)
    binv = (th2 / _Q).astype(jnp.int32)
    ok = (sums != jnp.float32(0.0)) & (binv < _P)
    vel_ref[...] = vel
    dist_ref[...] = dist
    th_ref[...] = th2
    bin_ref[...] = jnp.where(ok, binv, _P)


def _tc_features(nt2, bb=256):
    B = nt2.shape[0]
    p100 = lambda i: (i, 0)
    return pl.pallas_call(
        _feat_body,
        grid=(B // bb,),
        in_specs=[
            pl.BlockSpec((bb, _N * _W20), p100),
        ],
        out_specs=[pl.BlockSpec((bb, _N), p100)] * 4,
        out_shape=[
            jax.ShapeDtypeStruct((B, _N), jnp.float32),
            jax.ShapeDtypeStruct((B, _N), jnp.float32),
            jax.ShapeDtypeStruct((B, _N), jnp.float32),
            jax.ShapeDtypeStruct((B, _N), jnp.int32),
        ],
    )(nt2)


def _sc_stage(vel, dist, th, bins, B):
    apw = B // _NW
    chunks = apw // _A
    pw = _A * _N

    mesh = plsc.VectorSubcoreMesh(core_axis_name="c", subcore_axis_name="s")

    @functools.partial(
        pl.kernel,
        out_type=jax.ShapeDtypeStruct((B * 512,), jnp.float32),
        mesh=mesh,
        compiler_params=pltpu.CompilerParams(needs_layout_passes=False,
                                             use_tc_tiling_on_sc=True),
        scratch_types=[
            pltpu.VMEM((pw + _L,), jnp.float32),
            pltpu.VMEM((pw + _L,), jnp.float32),
            pltpu.VMEM((pw + _L,), jnp.float32),
            pltpu.VMEM((pw + _L,), jnp.int32),
            pltpu.VMEM((_A * 512,), jnp.float32),
        ],
    )
    def sck(vel_hbm, dist_hbm, th_hbm, bin_hbm, out_hbm,
            vbuf, dbuf, tbuf, bbuf, acc):
        wid = lax.axis_index("c") * _NS + lax.axis_index("s")
        base_agent = wid * apw

        lanes = lax.broadcasted_iota(jnp.int32, (_L,), 0)
        zeros16 = jnp.zeros((_L,), jnp.float32)
        ones16 = jnp.full((_L,), 1.0, jnp.float32)

        def chunk_body(ci, _):
            a0 = base_agent + ci * _A
            pltpu.sync_copy(vel_hbm.at[pl.ds(a0 * _N, pw)],
                            vbuf.at[pl.ds(0, pw)])
            pltpu.sync_copy(dist_hbm.at[pl.ds(a0 * _N, pw)],
                            dbuf.at[pl.ds(0, pw)])
            pltpu.sync_copy(th_hbm.at[pl.ds(a0 * _N, pw)],
                            tbuf.at[pl.ds(0, pw)])
            pltpu.sync_copy(bin_hbm.at[pl.ds(a0 * _N, pw)],
                            bbuf.at[pl.ds(0, pw)])

            def zero_body(zi, _):
                acc[pl.ds(zi * _L, _L)] = zeros16
                return 0
            lax.fori_loop(0, _A * 32, zero_body, 0)

            def agent_body(ai, _):
                abase = ai * _N
                arow = ai * 512
                for g in range((_N + _L - 1) // _L):
                    n0 = g * _L
                    idxb = abase + n0 + lanes
                    fvel = plsc.load_gather(vbuf, [idxb])
                    fdist = plsc.load_gather(dbuf, [idxb])
                    fth = plsc.load_gather(tbuf, [idxb])
                    binv = plsc.load_gather(bbuf, [idxb])
                    ok = binv < _P
                    if n0 + _L > _N:
                        ok = ok & (n0 + lanes < _N)
                    binc = jnp.where(ok, binv, 0)
                    row = arow + binc * (4 * _L) + lanes
                    plsc.addupdate_scatter(acc, [row], fvel, mask=ok)
                    plsc.addupdate_scatter(acc, [row + _L], fdist, mask=ok)
                    plsc.addupdate_scatter(acc, [row + 2 * _L], fth, mask=ok)
                    plsc.addupdate_scatter(acc, [row + 3 * _L], ones16,
                                           mask=ok)
                return 0

            lax.fori_loop(0, _A, agent_body, 0)
            pltpu.sync_copy(acc, out_hbm.at[pl.ds(a0 * 512, _A * 512)])
            return 0

        lax.fori_loop(0, chunks, chunk_body, 0)

    return sck(vel, dist, th, bins)


def _enc_body(s_ref, W_ref, b_ref, fsc_ref, sc_ref):
    s = s_ref[...]
    bb = s.shape[0]
    i512 = lax.broadcasted_iota(jnp.int32, (512, _P), 0)
    i8c = lax.broadcasted_iota(jnp.int32, (512, _P), 1)
    outs = []
    for f in range(4):
        sel = ((i512 // 64 == i8c) &
               ((i512 // _L) % 4 == f)).astype(jnp.float32)
        outs.append(lax.dot_general(s, sel, (((1,), (0,)), ((), ())),
                                    preferred_element_type=jnp.float32))
    n = outs[3] + jnp.float32(0.0001)
    v = outs[0] / n
    d = outs[1] / n
    g = outs[2] / n
    sc = jnp.stack([v, d, g], axis=-1)
    sc_ref[...] = sc
    flat = sc.reshape(bb * _P, 3)
    f = lax.dot_general(flat, W_ref[...], (((1,), (0,)), ((), ())),
                        preferred_element_type=jnp.float32)
    f = jnp.maximum(f + b_ref[...], 0.0)
    fsc_ref[...] = f.reshape(bb, _P, 128)


def _tc_stage(sums, W, b, bb=256):
    B = sums.shape[0]
    return pl.pallas_call(
        _enc_body,
        grid=(B // bb,),
        in_specs=[
            pl.BlockSpec((bb, 512), lambda i: (i, 0)),
            pl.BlockSpec((3, 128), lambda i: (0, 0)),
            pl.BlockSpec((1, 128), lambda i: (0, 0)),
        ],
        out_specs=[
            pl.BlockSpec((bb, _P, 128), lambda i: (i, 0, 0)),
            pl.BlockSpec((bb, _P, 3), lambda i: (i, 0, 0)),
        ],
        out_shape=[
            jax.ShapeDtypeStruct((B, _P, 128), jnp.float32),
            jax.ShapeDtypeStruct((B, _P, 3), jnp.float32),
        ],
    )(sums, W, b)


def kernel(trajs, nei_trajs, W, b):
    del trajs
    B = nei_trajs.shape[0]
    nt2 = nei_trajs.reshape(B, _N * _W20)
    h = B // 2
    b2 = b.reshape(1, 128)
    outs = []
    for lo in (0, h):
        vel, dist, th, bins = _tc_features(nt2[lo:lo + h])
        ps = _sc_stage(vel.reshape(h * _N), dist.reshape(h * _N),
                       th.reshape(h * _N), bins.reshape(h * _N),
                       h).reshape(h, 512)
        outs.append(_tc_stage(ps, W, b2))
    f_sc = jnp.concatenate([outs[0][0], outs[1][0]], axis=0)
    social_circle = jnp.concatenate([outs[0][1], outs[1][1]], axis=0)
    return (f_sc, social_circle)


# speedup vs baseline: 1.2195x; 1.2195x over previous
import functools

import jax
import jax.numpy as jnp
import numpy as np
from jax import lax
from jax.experimental import pallas as pl
from jax.experimental.pallas import tpu as pltpu
from jax.experimental.pallas import tpu_sc as plsc

_P = 8
_N = 100
_T = 10
_W20 = 2 * _T
_TWO_PI = np.float32(2.0 * np.pi)
_PI = np.float32(np.pi)
_HALF_PI = np.float32(np.pi / 2.0)
_Q = np.float32((2.0 * np.pi) / _P)

_ATAN_COEF = [np.float32(c) for c in (
    0.9999998999652773, -0.33332674305713506, 0.19987152762793234,
    -0.1417006414600397, 0.10531652562455919, -0.07302710404060626,
    0.040575162432038944, -0.01489037185530793, 0.0025799282931876833,
)]

_NC = 2
_NS = 16
_NW = _NC * _NS
_L = 16

_A = 64


def _atan2(a, b):
    aa = jnp.abs(a)
    ab = jnp.abs(b)
    mx = jnp.maximum(aa, ab)
    mn = jnp.minimum(aa, ab)
    den = jnp.where(mx > 0, mx, jnp.float32(1.0))
    t = mn / den
    z = t * t
    acc = _ATAN_COEF[-1]
    for c in _ATAN_COEF[-2::-1]:
        acc = acc * z + c
    r = t * acc
    r = jnp.where(aa > ab, _HALF_PI - r, r)
    r = jnp.where(b < 0, _PI - r, r)
    r = jnp.where(a < 0, -r, r)
    return r


def _feat_body(x_ref, vel_ref, dist_ref, th_ref, bin_ref):
    x = x_ref[...]
    r2000 = lax.broadcasted_iota(jnp.int32, (_N * _W20, _N), 0)
    c100 = lax.broadcasted_iota(jnp.int32, (_N * _W20, _N), 1)
    ssum = ((r2000 // _W20) == c100).astype(jnp.bfloat16)
    sums = lax.dot_general(x.astype(jnp.bfloat16), ssum,
                           (((1,), (0,)), ((), ())),
                           preferred_element_type=jnp.float32)
    r = lax.broadcasted_iota(jnp.int32, (_N * _W20, 512), 0)
    c = lax.broadcasted_iota(jnp.int32, (_N * _W20, 512), 1)
    g = c // 128
    j = c % 128
    i0 = j * _W20
    spos = (((g == 0) & (r == i0 + 18)) |
            ((g == 1) & (r == i0 + 19)) |
            ((g == 2) & (r == i0 + 18)) |
            ((g == 3) & (r == i0 + 19))).astype(jnp.float32)
    sneg = (((g == 0) & (r == i0)) |
            ((g == 1) & (r == i0 + 1))).astype(jnp.float32)
    sel = jnp.where(j < _N, spos - sneg, jnp.float32(0.0))
    y = lax.dot_general(x, sel, (((1,), (0,)), ((), ())),
                        preferred_element_type=jnp.float32)
    vx = y[:, 0:_N]
    vy = y[:, 128:128 + _N]
    lx = y[:, 256:256 + _N]
    ly = y[:, 384:384 + _N]
    vel = jnp.sqrt(vx * vx + vy * vy)
    dist = jnp.sqrt(lx * lx + ly * ly)
    th = _atan2(lx, ly)
    th2 = jnp.where(th < 0, th + _TWO_PI, th)
    binv = (th2 / _Q).astype(jnp.int32)
    ok = (sums != jnp.float32(0.0)) & (binv < _P)
    vel_ref[...] = vel
    dist_ref[...] = dist
    th_ref[...] = th2
    bin_ref[...] = jnp.where(ok, binv, _P)


def _tc_features(nt2, bb=256):
    B = nt2.shape[0]
    p100 = lambda i: (i, 0)
    return pl.pallas_call(
        _feat_body,
        grid=(B // bb,),
        in_specs=[
            pl.BlockSpec((bb, _N * _W20), p100),
        ],
        out_specs=[pl.BlockSpec((bb, _N), p100)] * 4,
        out_shape=[
            jax.ShapeDtypeStruct((B, _N), jnp.float32),
            jax.ShapeDtypeStruct((B, _N), jnp.float32),
            jax.ShapeDtypeStruct((B, _N), jnp.float32),
            jax.ShapeDtypeStruct((B, _N), jnp.int32),
        ],
    )(nt2)


def _sc_stage(vel, dist, th, bins, B):
    apw = B // _NW
    chunks = apw // _A
    pw = _A * _N

    mesh = plsc.VectorSubcoreMesh(core_axis_name="c", subcore_axis_name="s")

    @functools.partial(
        pl.kernel,
        out_type=jax.ShapeDtypeStruct((B * 512,), jnp.float32),
        mesh=mesh,
        compiler_params=pltpu.CompilerParams(needs_layout_passes=False,
                                             use_tc_tiling_on_sc=True),
        scratch_types=[
            pltpu.VMEM((pw + _L,), jnp.float32),
            pltpu.VMEM((pw + _L,), jnp.float32),
            pltpu.VMEM((pw + _L,), jnp.float32),
            pltpu.VMEM((pw + _L,), jnp.int32),
            pltpu.VMEM((_A * 512,), jnp.float32),
        ],
    )
    def sck(vel_hbm, dist_hbm, th_hbm, bin_hbm, out_hbm,
            vbuf, dbuf, tbuf, bbuf, acc):
        wid = lax.axis_index("c") * _NS + lax.axis_index("s")
        base_agent = wid * apw

        lanes = lax.broadcasted_iota(jnp.int32, (_L,), 0)
        zeros16 = jnp.zeros((_L,), jnp.float32)
        ones16 = jnp.full((_L,), 1.0, jnp.float32)

        def chunk_body(ci, _):
            a0 = base_agent + ci * _A
            pltpu.sync_copy(vel_hbm.at[pl.ds(a0 * _N, pw)],
                            vbuf.at[pl.ds(0, pw)])
            pltpu.sync_copy(dist_hbm.at[pl.ds(a0 * _N, pw)],
                            dbuf.at[pl.ds(0, pw)])
            pltpu.sync_copy(th_hbm.at[pl.ds(a0 * _N, pw)],
                            tbuf.at[pl.ds(0, pw)])
            pltpu.sync_copy(bin_hbm.at[pl.ds(a0 * _N, pw)],
                            bbuf.at[pl.ds(0, pw)])

            def zero_body(zi, _):
                for u in range(8):
                    acc[pl.ds((zi * 8 + u) * _L, _L)] = zeros16
                return 0
            lax.fori_loop(0, _A * 4, zero_body, 0)

            def agent_body(ai, _):
                abase = ai * _N
                arow = ai * 512
                for g in range((_N + _L - 1) // _L):
                    n0 = g * _L
                    idxb = abase + n0 + lanes
                    fvel = plsc.load_gather(vbuf, [idxb])
                    fdist = plsc.load_gather(dbuf, [idxb])
                    fth = plsc.load_gather(tbuf, [idxb])
                    binv = plsc.load_gather(bbuf, [idxb])
                    ok = binv < _P
                    if n0 + _L > _N:
                        ok = ok & (n0 + lanes < _N)
                    binc = jnp.where(ok, binv, 0)
                    row = arow + binc * (4 * _L) + lanes
                    plsc.addupdate_scatter(acc, [row], fvel, mask=ok)
                    plsc.addupdate_scatter(acc, [row + _L], fdist, mask=ok)
                    plsc.addupdate_scatter(acc, [row + 2 * _L], fth, mask=ok)
                    plsc.addupdate_scatter(acc, [row + 3 * _L], ones16,
                                           mask=ok)
                return 0

            lax.fori_loop(0, _A, agent_body, 0)
            pltpu.sync_copy(acc, out_hbm.at[pl.ds(a0 * 512, _A * 512)])
            return 0

        lax.fori_loop(0, chunks, chunk_body, 0)

    return sck(vel, dist, th, bins)


def _enc_body(s_ref, W_ref, b_ref, fsc_ref, sc_ref):
    s = s_ref[...]
    bb = s.shape[0]
    i512 = lax.broadcasted_iota(jnp.int32, (512, _P), 0)
    i8c = lax.broadcasted_iota(jnp.int32, (512, _P), 1)
    outs = []
    for f in range(4):
        sel = ((i512 // 64 == i8c) &
               ((i512 // _L) % 4 == f)).astype(jnp.float32)
        outs.append(lax.dot_general(s, sel, (((1,), (0,)), ((), ())),
                                    preferred_element_type=jnp.float32))
    n = outs[3] + jnp.float32(0.0001)
    v = outs[0] / n
    d = outs[1] / n
    g = outs[2] / n
    sc = jnp.stack([v, d, g], axis=-1)
    sc_ref[...] = sc
    flat = sc.reshape(bb * _P, 3)
    f = lax.dot_general(flat, W_ref[...], (((1,), (0,)), ((), ())),
                        preferred_element_type=jnp.float32)
    f = jnp.maximum(f + b_ref[...], 0.0)
    fsc_ref[...] = f.reshape(bb, _P, 128)


def _tc_stage(sums, W, b, bb=256):
    B = sums.shape[0]
    return pl.pallas_call(
        _enc_body,
        grid=(B // bb,),
        in_specs=[
            pl.BlockSpec((bb, 512), lambda i: (i, 0)),
            pl.BlockSpec((3, 128), lambda i: (0, 0)),
            pl.BlockSpec((1, 128), lambda i: (0, 0)),
        ],
        out_specs=[
            pl.BlockSpec((bb, _P, 128), lambda i: (i, 0, 0)),
            pl.BlockSpec((bb, _P, 3), lambda i: (i, 0, 0)),
        ],
        out_shape=[
            jax.ShapeDtypeStruct((B, _P, 128), jnp.float32),
            jax.ShapeDtypeStruct((B, _P, 3), jnp.float32),
        ],
    )(sums, W, b)


def kernel(trajs, nei_trajs, W, b):
    del trajs
    B = nei_trajs.shape[0]
    nt2 = nei_trajs.reshape(B, _N * _W20)
    vel, dist, th, bins = _tc_features(nt2)
    psums = _sc_stage(vel.reshape(B * _N), dist.reshape(B * _N),
                      th.reshape(B * _N), bins.reshape(B * _N),
                      B).reshape(B, 512)
    f_sc, social_circle = _tc_stage(psums, W, b.reshape(1, 128))
    return (f_sc, social_circle)
